# baseline (speedup 1.0000x reference)
import functools

import jax
import jax.numpy as jnp
from jax import lax
from jax.experimental import pallas as pl
from jax.experimental.pallas import tpu as pltpu
from jax.experimental.pallas import tpu_sc as plsc

M = 1000000
D = 64
HD = D // 2
B = 16384
NT = 16
BPT = B // NT
CH = 128
NCH = BPT // CH
WAVE = 2


def _sc_body(idx_hbm, mem_hbm, val_hbm, out_hbm, tbl_sh, acc_sh,
             idx_v, jv_v, rep_v, rows_v, half_v, sem):
    cid = lax.axis_index("c")
    sid = lax.axis_index("s")

    @pl.when(cid == 0)
    def _work():
        base = sid * BPT
        pltpu.sync_copy(idx_hbm.at[sid], idx_v)

        for c in range(NCH):
            for g in range(CH // 16):
                jv_v[c, pl.ds(g * 16, 16)] = (
                    base + c * CH + g * 16 + lax.iota(jnp.int32, 16))
        for c in range(NCH):
            pltpu.sync_copy(jv_v.at[c], tbl_sh.at[idx_v.at[c]])
        plsc.subcore_barrier()

        for c in range(NCH):
            pltpu.sync_copy(tbl_sh.at[idx_v.at[c]], rep_v.at[c])

        for h in range(2):
            cols = pl.ds(h * HD, HD)

            for w in range(NCH // WAVE):
                gathers = [
                    pltpu.async_copy(mem_hbm.at[idx_v.at[w * WAVE + k]],
                                     rows_v.at[k], sem)
                    for k in range(WAVE)
                ]
                for gth in gathers:
                    gth.wait()
                for k in range(WAVE):
                    def _compact(r, _, k=k):
                        for g in range(HD // 16):
                            half_v[r, pl.ds(g * 16, 16)] = (
                                rows_v[k, r, pl.ds(h * HD + g * 16, 16)])
                        return _
                    lax.fori_loop(0, CH, _compact, None, unroll=4)
                    pltpu.sync_copy(half_v,
                                    acc_sh.at[rep_v.at[w * WAVE + k]])
            plsc.subcore_barrier()

            for c in range(NCH):
                pltpu.sync_copy(val_hbm.at[pl.ds(base + c * CH, CH), cols],
                                half_v)
                pltpu.sync_copy(half_v, acc_sh.at[rep_v.at[c]], add=True)
            plsc.subcore_barrier()

            for c in range(NCH):
                pltpu.sync_copy(acc_sh.at[rep_v.at[c]], half_v)
                pltpu.sync_copy(half_v,
                                out_hbm.at[pl.ds(base + c * CH, CH), cols])
            plsc.subcore_barrier()


@functools.partial(
    pl.kernel,
    out_type=jax.ShapeDtypeStruct((B, D), jnp.float32),
    mesh=plsc.VectorSubcoreMesh(core_axis_name="c", subcore_axis_name="s",
                                num_cores=2, num_subcores=16),
    scratch_types=[
        pltpu.VMEM_SHARED((M,), jnp.int32),
        pltpu.VMEM_SHARED((B, HD), jnp.float32),
        pltpu.VMEM((NCH, CH), jnp.int32),
        pltpu.VMEM((NCH, CH), jnp.int32),
        pltpu.VMEM((NCH, CH), jnp.int32),
        pltpu.VMEM((WAVE, CH, D), jnp.float32),
        pltpu.VMEM((CH, HD), jnp.float32),
        pltpu.SemaphoreType.DMA,
    ],
    compiler_params=pltpu.CompilerParams(use_tc_tiling_on_sc=False),
)
def _sc_learner(idx_hbm, mem_hbm, val_hbm, out_hbm, *rest):
    _sc_body(idx_hbm, mem_hbm, val_hbm, out_hbm, *rest)


def kernel(mem, val, idx):
    idx3 = idx.astype(jnp.int32).reshape(NT, NCH, CH)
    return _sc_learner(idx3, mem, val)


# speedup vs baseline: 1.0312x; 1.0312x over previous
import functools

import jax
import jax.numpy as jnp
from jax import lax
from jax.experimental import pallas as pl
from jax.experimental.pallas import tpu as pltpu
from jax.experimental.pallas import tpu_sc as plsc

M = 1000000
D = 64
HD = D // 2
B = 16384
NT = 16
BPT = B // NT
CH = 128
NCH = BPT // CH
WAVE = 4


def _sc_body(idx_hbm, mem_hbm, val_hbm, out_hbm, tbl_sh, acc_sh,
             idx_v, jv_v, rep_v, gi_v, oi_v, rows_v, vch_v, sem):
    cid = lax.axis_index("c")
    sid = lax.axis_index("s")

    @pl.when(cid == 0)
    def _work():
        base = sid * BPT
        pltpu.sync_copy(idx_hbm.at[sid], idx_v)

        for c in range(NCH):
            for g in range(CH // 16):
                jv_v[c, pl.ds(g * 16, 16)] = (
                    base + c * CH + g * 16 + lax.iota(jnp.int32, 16))
        for c in range(NCH):
            pltpu.sync_copy(jv_v.at[c], tbl_sh.at[idx_v.at[c]])
        plsc.subcore_barrier()

        for c in range(NCH):
            pltpu.sync_copy(tbl_sh.at[idx_v.at[c]], rep_v.at[c])

        for h in range(2):
            for c in range(NCH):
                for g in range(CH // 16):
                    s = pl.ds(g * 16, 16)
                    gi_v[c, s] = idx_v[c, s] * 2 + h
                    oi_v[c, s] = jv_v[c, s] * 2 + h

            for w in range(NCH // WAVE):
                gathers = [
                    pltpu.async_copy(mem_hbm.at[gi_v.at[w * WAVE + k]],
                                     rows_v.at[k], sem)
                    for k in range(WAVE)
                ]
                for gth in gathers:
                    gth.wait()
                for k in range(WAVE):
                    pltpu.sync_copy(rows_v.at[k],
                                    acc_sh.at[rep_v.at[w * WAVE + k]])
            plsc.subcore_barrier()

            for c in range(NCH):
                pltpu.sync_copy(val_hbm.at[oi_v.at[c]], vch_v)
                pltpu.sync_copy(vch_v, acc_sh.at[rep_v.at[c]], add=True)
            plsc.subcore_barrier()

            for c in range(NCH):
                pltpu.sync_copy(acc_sh.at[rep_v.at[c]], vch_v)
                pltpu.sync_copy(vch_v, out_hbm.at[oi_v.at[c]])
            plsc.subcore_barrier()


@functools.partial(
    pl.kernel,
    out_type=jax.ShapeDtypeStruct((2 * B, HD), jnp.float32),
    mesh=plsc.VectorSubcoreMesh(core_axis_name="c", subcore_axis_name="s",
                                num_cores=2, num_subcores=16),
    scratch_types=[
        pltpu.VMEM_SHARED((M,), jnp.int32),
        pltpu.VMEM_SHARED((B, HD), jnp.float32),
        pltpu.VMEM((NCH, CH), jnp.int32),
        pltpu.VMEM((NCH, CH), jnp.int32),
        pltpu.VMEM((NCH, CH), jnp.int32),
        pltpu.VMEM((NCH, CH), jnp.int32),
        pltpu.VMEM((NCH, CH), jnp.int32),
        pltpu.VMEM((WAVE, CH, HD), jnp.float32),
        pltpu.VMEM((CH, HD), jnp.float32),
        pltpu.SemaphoreType.DMA,
    ],
    compiler_params=pltpu.CompilerParams(use_tc_tiling_on_sc=False),
)
def _sc_learner(idx_hbm, mem_hbm, val_hbm, out_hbm, *rest):
    _sc_body(idx_hbm, mem_hbm, val_hbm, out_hbm, *rest)


def kernel(mem, val, idx):
    idx3 = idx.astype(jnp.int32).reshape(NT, NCH, CH)
    mem2 = mem.reshape(2 * M, HD)
    val2 = val.reshape(2 * B, HD)
    out2 = _sc_learner(idx3, mem2, val2)
    return out2.reshape(B, D)
